# baseline (speedup 1.0000x reference)
import functools

import jax
import jax.numpy as jnp
from jax import lax
from jax.experimental import pallas as pl
from jax.experimental.pallas import tpu as pltpu
from jax.experimental.pallas import tpu_sc as plsc

NUM_EMBEDDINGS = 1000000
EMB_D = 64
TSTEPS = 10
BATCH_B = 1024
SEQ_S = 20
N_TOK = BATCH_B * SEQ_S
NUM_WORKERS = 32
PER_W = N_TOK // NUM_WORKERS
GCHUNK = 128
NG = PER_W // GCHUNK
LANES = 16

_mesh = plsc.VectorSubcoreMesh(core_axis_name="c", subcore_axis_name="s")


@functools.partial(
    pl.kernel,
    mesh=_mesh,
    compiler_params=pltpu.CompilerParams(use_tc_tiling_on_sc=False),
    out_type=jax.ShapeDtypeStruct((N_TOK, TSTEPS, EMB_D), jnp.float32),
    scratch_types=[
        pltpu.VMEM((NG, GCHUNK), jnp.int32),
        pltpu.VMEM((PER_W, EMB_D), jnp.float32),
        pltpu.VMEM((LANES,), jnp.float32),
        pltpu.SemaphoreType.DMA,
    ],
)
def _spike_embed(ids_hbm, table_hbm, thr_hbm, out_hbm, idx_v, rows_v, thr_v, sem):
    wid = lax.axis_index("s") * 2 + lax.axis_index("c")
    base = wid * PER_W

    pltpu.sync_copy(thr_hbm, thr_v)
    pltpu.sync_copy(ids_hbm.at[wid], idx_v)

    copies = []
    for j in range(NG):
        copies.append(
            pltpu.async_copy(
                table_hbm.at[idx_v.at[j]],
                rows_v.at[pl.ds(j * GCHUNK, GCHUNK)],
                sem,
            )
        )
    for c in copies:
        c.wait()

    thr = thr_v[...]

    def body(i, carry):
        for c in range(EMB_D // LANES):
            x = rows_v[i, pl.ds(c * LANES, LANES)]
            rows_v[i, pl.ds(c * LANES, LANES)] = jnp.where(
                x > thr, jnp.float32(1.0), jnp.float32(0.0)
            )
        return carry

    lax.fori_loop(0, PER_W, body, 0)

    for t in range(TSTEPS):
        pltpu.sync_copy(rows_v, out_hbm.at[pl.ds(base, PER_W), t])


def kernel(token_ids, W, adaptive_threshold):
    ids = token_ids.reshape(NUM_WORKERS, NG, GCHUNK).astype(jnp.int32)
    thr16 = jnp.broadcast_to(
        adaptive_threshold.astype(jnp.float32), (LANES,)
    )
    out = _spike_embed(ids, W, thr16)
    return out.reshape(BATCH_B, SEQ_S, TSTEPS, EMB_D)


# speedup vs baseline: 1.1710x; 1.1710x over previous
import functools

import jax
import jax.numpy as jnp
from jax import lax
from jax.experimental import pallas as pl
from jax.experimental.pallas import tpu as pltpu
from jax.experimental.pallas import tpu_sc as plsc

NUM_EMBEDDINGS = 1000000
EMB_D = 64
TSTEPS = 10
BATCH_B = 1024
SEQ_S = 20
NUM_WORKERS = 32
BBLK = 128
NBB = BATCH_B // BBLK
UNITS = SEQ_S * NBB
UNITS_PER_W = UNITS // NUM_WORKERS
LANES = 16

_mesh = plsc.VectorSubcoreMesh(core_axis_name="c", subcore_axis_name="s")


@functools.partial(
    pl.kernel,
    mesh=_mesh,
    compiler_params=pltpu.CompilerParams(
        use_tc_tiling_on_sc=False, needs_layout_passes=False
    ),
    out_type=jax.ShapeDtypeStruct(
        (SEQ_S, TSTEPS, EMB_D // 8, NBB, 8, BBLK), jnp.float32
    ),
    scratch_types=[
        pltpu.VMEM((BBLK,), jnp.int32),
        pltpu.VMEM((BBLK, EMB_D), jnp.float32),
        pltpu.VMEM((EMB_D // 8, 8, BBLK), jnp.float32),
        pltpu.VMEM((LANES,), jnp.float32),
        pltpu.SemaphoreType.DMA,
    ],
)
def _spike_embed(ids_hbm, table_hbm, thr_hbm, out_hbm, idx_v, rows_v, asm_v, thr_v, sem):
    wid = lax.axis_index("s") * 2 + lax.axis_index("c")

    pltpu.sync_copy(thr_hbm, thr_v)
    thr = thr_v[...]
    lane = lax.iota(jnp.int32, LANES)
    one = jnp.float32(1.0)
    zero = jnp.float32(0.0)

    def do_unit(u, carry):
        unit = wid * UNITS_PER_W + u
        s = unit // NBB
        b_blk = unit % NBB

        pltpu.sync_copy(ids_hbm.at[s, b_blk], idx_v)
        pltpu.async_copy(table_hbm.at[idx_v], rows_v, sem).wait()

        def do_d(d, carry_d):
            db = d // 8
            di = d - db * 8
            dvec = jnp.broadcast_to(d, (LANES,))
            for bc in range(BBLK // LANES):
                x = plsc.load_gather(rows_v, [bc * LANES + lane, dvec])
                asm_v[db, di, pl.ds(bc * LANES, LANES)] = jnp.where(
                    x > thr, one, zero
                )
            return carry_d

        lax.fori_loop(0, EMB_D, do_d, 0)

        for t in range(TSTEPS):
            pltpu.sync_copy(asm_v, out_hbm.at[s, t, :, b_blk])
        return carry

    lax.fori_loop(0, UNITS_PER_W, do_unit, 0)


def kernel(token_ids, W, adaptive_threshold):
    ids = token_ids.astype(jnp.int32).T.reshape(SEQ_S, NBB, BBLK)
    thr16 = jnp.broadcast_to(adaptive_threshold.astype(jnp.float32), (LANES,))
    out6 = _spike_embed(ids, W, thr16)
    return out6.transpose(3, 5, 0, 1, 2, 4).reshape(
        BATCH_B, SEQ_S, TSTEPS, EMB_D
    )


# speedup vs baseline: 1.7211x; 1.4698x over previous
import functools

import jax
import jax.numpy as jnp
from jax import lax
from jax.experimental import pallas as pl
from jax.experimental.pallas import tpu as pltpu
from jax.experimental.pallas import tpu_sc as plsc

NUM_EMBEDDINGS = 1000000
EMB_D = 64
TSTEPS = 10
BATCH_B = 1024
SEQ_S = 20
NUM_WORKERS = 32
BBLK = 128
NBB = BATCH_B // BBLK
UNITS = SEQ_S * NBB
UNITS_PER_W = UNITS // NUM_WORKERS
LANES = 16

_mesh = plsc.VectorSubcoreMesh(core_axis_name="c", subcore_axis_name="s")


@functools.partial(
    pl.kernel,
    mesh=_mesh,
    compiler_params=pltpu.CompilerParams(
        use_tc_tiling_on_sc=True, needs_layout_passes=False
    ),
    out_type=jax.ShapeDtypeStruct(
        (SEQ_S, TSTEPS, EMB_D // 8, NBB, 8, BBLK), jnp.float32
    ),
    scratch_types=[
        pltpu.VMEM((BBLK,), jnp.int32),
        pltpu.VMEM((BBLK // 2, 8, EMB_D), jnp.float32),
        pltpu.VMEM((2, EMB_D // 8, 8, BBLK), jnp.float32),
        pltpu.VMEM((LANES,), jnp.float32),
        pltpu.SemaphoreType.DMA,
        pltpu.SemaphoreType.DMA,
    ],
)
def _spike_embed(
    ids_hbm, w_hbm, thr_hbm, out_hbm, idx_v, win_v, asm_v, thr_v, sem_g, sem_o
):
    wid = lax.axis_index("s") * 2 + lax.axis_index("c")

    pltpu.sync_copy(thr_hbm, thr_v)
    thr = thr_v[...]
    lane = lax.iota(jnp.int32, LANES)
    one = jnp.float32(1.0)
    zero = jnp.float32(0.0)

    out_handles = []
    for u in range(UNITS_PER_W):
        buf = u % 2
        unit = wid * UNITS_PER_W + u
        s = unit // NBB
        b_blk = unit - s * NBB

        pltpu.sync_copy(ids_hbm.at[s, b_blk], idx_v)

        if u >= 2:
            for h in out_handles[u - 2]:
                h.wait()

        for half in range(2):
            hb = half * (BBLK // 2)

            def fire(bc, carry):
                idvec = idx_v[pl.ds(hb + bc * LANES, LANES)]
                base8 = idvec & jnp.int32(~7)
                for l in range(LANES):
                    base = pl.multiple_of(base8[l], 8)
                    pltpu.make_async_copy(
                        w_hbm.at[pl.ds(base, 8), :],
                        win_v.at[bc * LANES + l],
                        sem_g,
                    ).start()
                return carry

            lax.fori_loop(0, BBLK // (2 * LANES), fire, 0)

            def drain(b, carry):
                pltpu.make_async_copy(
                    w_hbm.at[pl.ds(0, 8), :],
                    win_v.at[b],
                    sem_g,
                ).wait()
                return carry

            lax.fori_loop(0, BBLK // 2, drain, 0)

            def do_d(d, carry_d):
                db = d // 8
                di = d - db * 8
                dvec = jnp.broadcast_to(d, (LANES,))
                for bc in range(BBLK // (2 * LANES)):
                    idvec = idx_v[pl.ds(hb + bc * LANES, LANES)]
                    rowsel = idvec & 7
                    x = plsc.load_gather(
                        win_v, [bc * LANES + lane, rowsel, dvec]
                    )
                    asm_v[
                        buf, db, di, pl.ds(hb + bc * LANES, LANES)
                    ] = jnp.where(x > thr, one, zero)
                return carry_d

            lax.fori_loop(0, EMB_D, do_d, 0)

        hs = []
        for t in range(TSTEPS):
            c = pltpu.make_async_copy(
                asm_v.at[buf], out_hbm.at[s, t, :, b_blk], sem_o
            )
            c.start()
            hs.append(c)
        out_handles.append(hs)

    for hs in out_handles[-2:]:
        for h in hs:
            h.wait()


def kernel(token_ids, W, adaptive_threshold):
    ids = token_ids.astype(jnp.int32).T.reshape(SEQ_S, NBB, BBLK)
    thr16 = jnp.broadcast_to(adaptive_threshold.astype(jnp.float32), (LANES,))
    out6 = _spike_embed(ids, W, thr16)
    return out6.transpose(3, 5, 0, 1, 2, 4).reshape(
        BATCH_B, SEQ_S, TSTEPS, EMB_D
    )
